# baseline (speedup 1.0000x reference)
import jax
import jax.numpy as jnp
from jax.experimental import pallas as pl

_L2 = 1.0
_LINF = 0.02
_K = 2048


def _body(y_ref, yh_ref, mask_ref, out_ref):
    B, N = y_ref.shape
    m = mask_ref[...]
    d = y_ref[...] * m - yh_ref[...] * m
    sq = d * d
    total = jnp.sum(sq)

    bits = jax.lax.bitcast_convert_type(sq, jnp.int32)

    def count_ge(mid):
        lt = jax.lax.shift_right_logical(bits - mid, 31)
        nsub = 2
        w = N // nsub
        parts = [jnp.sum(lt[:, i * w:(i + 1) * w], axis=1, keepdims=True)
                 for i in range(nsub)]
        while len(parts) > 1:
            parts = [parts[i] + parts[i + 1] for i in range(0, len(parts), 2)]
        return N - parts[0]

    row_max = jnp.max(bits, axis=1, keepdims=True)
    lo = jnp.zeros((B, 1), jnp.int32)
    hi = row_max
    c_lo = jnp.full((B, 1), jnp.int32(N))
    c_hi1 = jnp.ones((B, 1), jnp.float32)

    def _done(lo, hi, c_lo):
        return (c_lo == _K) | (lo >= hi)

    def cond(carry):
        i, lo, hi, c_lo, c_hi1 = carry
        return jnp.any(~_done(lo, hi, c_lo))

    def step(carry):
        i, lo, hi, c_lo, c_hi1 = carry
        frac = jnp.maximum((c_lo - _K).astype(jnp.float32), 0.0) / (
            jnp.maximum(c_lo.astype(jnp.float32) - c_hi1, 1.0))
        m_interp = lo + (frac * (hi + 1 - lo).astype(jnp.float32)).astype(
            jnp.int32)
        m_bisect = lo + ((hi - lo + 1) >> 1)
        mid = jnp.where(i % 3 == 2, m_bisect, m_interp)
        mid = jnp.clip(mid, lo + 1, hi)
        cnt = count_ge(mid)
        upd = ~_done(lo, hi, c_lo)
        ge = cnt >= _K
        lo = jnp.where(upd & ge, mid, lo)
        hi = jnp.where(upd & ~ge, mid - 1, hi)
        c_lo = jnp.where(upd & ge, cnt, c_lo)
        c_hi1 = jnp.where(upd & ~ge, cnt.astype(jnp.float32), c_hi1)
        return i + 1, lo, hi, c_lo, c_hi1

    _, lo, hi, c_lo, c_hi1 = jax.lax.while_loop(
        cond, step, (jnp.int32(0), lo, hi, c_lo, c_hi1))

    sentinel = jnp.int32(0x7FFFFFFF)
    ge_lo = bits >= lo
    t_bits = jnp.min(jnp.where(ge_lo, bits, sentinel), axis=1, keepdims=True)
    t = jax.lax.bitcast_convert_type(t_bits, jnp.float32)

    gt = bits > t_bits
    s_gt = jnp.sum(jnp.where(gt, sq, 0.0), axis=1, keepdims=True)
    c_gt = jnp.sum(gt.astype(jnp.int32), axis=1, keepdims=True)
    topk_sum = s_gt + (_K - c_gt).astype(jnp.float32) * t

    linf = jnp.sum(topk_sum) / B
    l2 = total / (B * N)
    out_ref[...] = jnp.reshape(_L2 * l2 + _LINF * linf, (1, 1))


@jax.jit
def kernel(y, yh, mask):
    res = pl.pallas_call(
        _body,
        out_shape=jax.ShapeDtypeStruct((1, 1), jnp.float32),
    )(y, yh, mask)
    return res[0, 0]


# speedup vs baseline: 1.0413x; 1.0413x over previous
import jax
import jax.numpy as jnp
from jax.experimental import pallas as pl

_L2 = 1.0
_LINF = 0.02
_K = 2048


def _body(y_ref, yh_ref, mask_ref, out_ref):
    B, N = y_ref.shape
    m = mask_ref[...]
    d = y_ref[...] * m - yh_ref[...] * m
    sq = d * d
    total = jnp.sum(sq)

    bits = jax.lax.bitcast_convert_type(sq, jnp.int32)

    def count_ge(mid):
        lt = jax.lax.shift_right_logical(bits - mid, 31)
        nsub = 4
        w = N // nsub
        parts = [jnp.sum(lt[:, i * w:(i + 1) * w], axis=1, keepdims=True)
                 for i in range(nsub)]
        while len(parts) > 1:
            parts = [parts[i] + parts[i + 1] for i in range(0, len(parts), 2)]
        return N - parts[0]

    row_max = jnp.max(bits, axis=1, keepdims=True)
    lo = jnp.zeros((B, 1), jnp.int32)
    hi = row_max
    c_lo = jnp.full((B, 1), jnp.int32(N))
    c_hi1 = jnp.ones((B, 1), jnp.float32)

    def _done(lo, hi, c_lo):
        return (c_lo == _K) | (lo >= hi)

    def cond(carry):
        i, lo, hi, c_lo, c_hi1 = carry
        return jnp.any(~_done(lo, hi, c_lo))

    def step(carry):
        i, lo, hi, c_lo, c_hi1 = carry
        frac = jnp.maximum((c_lo - _K).astype(jnp.float32), 0.0) / (
            jnp.maximum(c_lo.astype(jnp.float32) - c_hi1, 1.0))
        m_interp = lo + (frac * (hi + 1 - lo).astype(jnp.float32)).astype(
            jnp.int32)
        m_bisect = lo + ((hi - lo + 1) >> 1)
        mid = jnp.where(i % 3 == 2, m_bisect, m_interp)
        mid = jnp.clip(mid, lo + 1, hi)
        cnt = count_ge(mid)
        upd = ~_done(lo, hi, c_lo)
        ge = cnt >= _K
        lo = jnp.where(upd & ge, mid, lo)
        hi = jnp.where(upd & ~ge, mid - 1, hi)
        c_lo = jnp.where(upd & ge, cnt, c_lo)
        c_hi1 = jnp.where(upd & ~ge, cnt.astype(jnp.float32), c_hi1)
        return i + 1, lo, hi, c_lo, c_hi1

    _, lo, hi, c_lo, c_hi1 = jax.lax.while_loop(
        cond, step, (jnp.int32(0), lo, hi, c_lo, c_hi1))

    sentinel = jnp.int32(0x7FFFFFFF)
    ge_lo = bits >= lo
    t_bits = jnp.min(jnp.where(ge_lo, bits, sentinel), axis=1, keepdims=True)
    t = jax.lax.bitcast_convert_type(t_bits, jnp.float32)

    gt = bits > t_bits
    s_gt = jnp.sum(jnp.where(gt, sq, 0.0), axis=1, keepdims=True)
    c_gt = jnp.sum(gt.astype(jnp.int32), axis=1, keepdims=True)
    topk_sum = s_gt + (_K - c_gt).astype(jnp.float32) * t

    linf = jnp.sum(topk_sum) / B
    l2 = total / (B * N)
    out_ref[...] = jnp.reshape(_L2 * l2 + _LINF * linf, (1, 1))


@jax.jit
def kernel(y, yh, mask):
    res = pl.pallas_call(
        _body,
        out_shape=jax.ShapeDtypeStruct((1, 1), jnp.float32),
    )(y, yh, mask)
    return res[0, 0]
